# Initial kernel scaffold; baseline (speedup 1.0000x reference)
import jax
import jax.numpy as jnp
from jax import lax
from jax.experimental import pallas as pl
from jax.experimental.pallas import tpu as pltpu, tpu_sc as plsc

N_USERS = 5000
N_ITEMS = 5000
N = N_USERS + N_ITEMS
D = 256
DH = 128
N_HOPS = 3
NNZ = 160000
NC = 2
NS = 16
L = 16

EPT = NNZ // NS
E = 100
NCHUNK = EPT // E
R = 80
ROWS_BIG = 640
COEF_PAD = ROWS_BIG * NS


def _scale_rows(buf, nrows, coef_ref, base):
    def body(i, _):
        splat = plsc.load_gather(coef_ref, [jnp.full((L,), base + i, jnp.int32)])
        for j in range(DH // L):
            sl = pl.ds(j * L, L)
            buf[i, sl] = buf[i, sl] * splat
        return 0
    lax.fori_loop(0, nrows, body, 0)


def _graph_conv(agg0, coefs, erows, ecols, evals):
    mesh = plsc.VectorSubcoreMesh(core_axis_name="c", subcore_axis_name="s",
                                  num_cores=NC, num_subcores=NS)

    def body(agg0_hbm, coef_hbm, erows_hbm, ecols_hbm, evals_hbm,
             out_hbm, scr1_hbm, scr2_hbm,
             acc, cols_v, vals_v, rows_v, gbuf, rbuf, zbuf, coef_v, sem):
        c = lax.axis_index("c")
        s = lax.axis_index("s")

        ebase = s * EPT
        pltpu.sync_copy(ecols_hbm.at[pl.ds(ebase, EPT)], cols_v)
        pltpu.sync_copy(evals_hbm.at[pl.ds(ebase, EPT)], vals_v)

        row_base = s * ROWS_BIG
        nrows = jnp.where(s < NS - 1, ROWS_BIG, N - (NS - 1) * ROWS_BIG)
        nrc = nrows // R

        def zb(i, _):
            for j in range(DH // L):
                zbuf[i, pl.ds(j * L, L)] = jnp.zeros((L,), jnp.float32)
            return 0
        lax.fori_loop(0, R, zb, 0)

        def zero_acc(j, _):
            pltpu.sync_copy(zbuf, acc.at[pl.ds(row_base + j * R, R)])
            return 0
        lax.fori_loop(0, nrc, zero_acc, 0)

        pltpu.sync_copy(coef_hbm.at[0, pl.ds(row_base, ROWS_BIG)], coef_v)

        def ego(j, _):
            rb = row_base + j * R
            pltpu.sync_copy(agg0_hbm.at[c, pl.ds(rb, R)], rbuf)
            _scale_rows(rbuf, R, coef_v, j * R)
            pltpu.sync_copy(rbuf, out_hbm.at[pl.ds(rb, R), 0, pl.ds(c * DH, DH)])
            return 0
        lax.fori_loop(0, nrc, ego, 0)

        plsc.subcore_barrier()

        hop_srcs = [agg0_hbm, scr1_hbm, scr2_hbm]
        hop_raws = [scr1_hbm, scr2_hbm, None]

        for h in range(1, N_HOPS + 1):
            src = hop_srcs[h - 1]
            raw = hop_raws[h - 1]

            def edge_chunk(i, _, src=src):
                eb = ebase + i * E
                pltpu.sync_copy(erows_hbm.at[pl.ds(eb, E)], rows_v)
                idx = cols_v.at[pl.ds(i * E, E)]
                pltpu.async_copy(src.at[c].at[idx], gbuf, sem).wait()

                def scale(e, _):
                    splat = plsc.load_gather(
                        vals_v, [jnp.full((L,), i * E + e, jnp.int32)])
                    for j in range(DH // L):
                        sl = pl.ds(j * L, L)
                        gbuf[e, sl] = gbuf[e, sl] * splat
                    return 0
                lax.fori_loop(0, E, scale, 0)

                pltpu.sync_copy(gbuf, acc.at[rows_v], add=True)
                return 0
            lax.fori_loop(0, NCHUNK, edge_chunk, 0)

            plsc.subcore_barrier()

            pltpu.sync_copy(coef_hbm.at[h, pl.ds(row_base, ROWS_BIG)], coef_v)

            def drain(j, _, raw=raw, h=h):
                rb = row_base + j * R
                pltpu.sync_copy(acc.at[pl.ds(rb, R)], rbuf)
                if raw is not None:
                    pltpu.sync_copy(rbuf, raw.at[c, pl.ds(rb, R)])
                    pltpu.sync_copy(zbuf, acc.at[pl.ds(rb, R)])
                _scale_rows(rbuf, R, coef_v, j * R)
                pltpu.sync_copy(rbuf, out_hbm.at[pl.ds(rb, R), h, pl.ds(c * DH, DH)])
                return 0
            lax.fori_loop(0, nrc, drain, 0)

            plsc.subcore_barrier()

    f = pl.kernel(
        body,
        out_type=(
            jax.ShapeDtypeStruct((N, N_HOPS + 1, D), jnp.float32),
            jax.ShapeDtypeStruct((NC, N, DH), jnp.float32),
            jax.ShapeDtypeStruct((NC, N, DH), jnp.float32),
        ),
        mesh=mesh,
        scratch_types=[
            pltpu.VMEM_SHARED((N, DH), jnp.float32),
            pltpu.VMEM((EPT,), jnp.int32),
            pltpu.VMEM((EPT,), jnp.float32),
            pltpu.VMEM((E,), jnp.int32),
            pltpu.VMEM((E, DH), jnp.float32),
            pltpu.VMEM((R, DH), jnp.float32),
            pltpu.VMEM((R, DH), jnp.float32),
            pltpu.VMEM((ROWS_BIG,), jnp.float32),
            pltpu.SemaphoreType.DMA,
        ],
    )
    out, _, _ = f(agg0, coefs, erows, ecols, evals)
    return out


def kernel(user_embed, item_embed, user_t, item_t, edge_rows, edge_cols, edge_vals):
    all_embed = jnp.concatenate([user_embed, item_embed], axis=0)
    agg0 = all_embed.reshape(N, NC, DH).transpose(1, 0, 2)

    t = jnp.concatenate([user_t[:, 0], item_t[:, 0]])
    decay = 1.0 - t
    coefs = jnp.stack([t, t * decay, t * decay**2, t * decay**3])
    coefs = jnp.pad(coefs, ((0, 0), (0, COEF_PAD - N)))

    out = _graph_conv(agg0, coefs, edge_rows, edge_cols, edge_vals)
    return out[:N_USERS], out[N_USERS:]


# speedup vs baseline: 2.7149x; 2.7149x over previous
import jax
import jax.numpy as jnp
import numpy as np
from jax import lax
from jax.experimental import pallas as pl
from jax.experimental.pallas import tpu as pltpu, tpu_sc as plsc

N_USERS = 5000
N_ITEMS = 5000
N = N_USERS + N_ITEMS
D = 256
DH = 128
N_HOPS = 3
NNZ = 160000
NC = 2
NS = 16
L = 16

EPT = NNZ // NS
E = 80
NCHUNK = EPT // E
R = 80
ROWS_BIG = 640
COEF_PAD = ROWS_BIG * NS


def _splat(vec16, zeros16, lane):
    return vec16.at[zeros16 + lane].get(mode="promise_in_bounds")


def _scale_rows(buf, nrows, coef_ref, base):
    zeros16 = lax.broadcasted_iota(jnp.int32, (L,), 0) * 0

    def body(g, _):
        c16 = coef_ref[pl.ds(base + g * L, L)]
        for e in range(L):
            splat = _splat(c16, zeros16, e)
            row = g * L + e
            for j in range(DH // L):
                sl = pl.ds(j * L, L)
                buf[row, sl] = buf[row, sl] * splat
        return 0
    lax.fori_loop(0, nrows // L, body, 0)


def _graph_conv(agg0, coefs, erows, ecols, evals):
    mesh = plsc.VectorSubcoreMesh(core_axis_name="c", subcore_axis_name="s",
                                  num_cores=NC, num_subcores=NS)

    def body(agg0_hbm, coef_hbm, erows_hbm, ecols_hbm, evals_hbm,
             out_hbm, scr1_hbm, scr2_hbm,
             acc, cols_v, vals_v, rows_v, gbuf, rbuf, zbuf, coef_v, sem):
        c = lax.axis_index("c")
        s = lax.axis_index("s")

        ebase = s * EPT
        row_base = s * ROWS_BIG
        nrows = jnp.where(s < NS - 1, ROWS_BIG, N - (NS - 1) * ROWS_BIG)
        nrc = nrows // R

        def zb(i, _):
            for j in range(DH // L):
                zbuf[i, pl.ds(j * L, L)] = jnp.zeros((L,), jnp.float32)
            return 0
        lax.fori_loop(0, R, zb, 0)

        def zero_acc(j, _):
            pltpu.sync_copy(zbuf, acc.at[pl.ds(row_base + j * R, R)])
            return 0
        lax.fori_loop(0, nrc, zero_acc, 0)

        pltpu.sync_copy(coef_hbm.at[0, pl.ds(row_base, ROWS_BIG)], coef_v)

        def ego(j, _):
            rb = row_base + j * R
            pltpu.sync_copy(agg0_hbm.at[c, pl.ds(rb, R)], rbuf)
            _scale_rows(rbuf, R, coef_v, j * R)
            pltpu.sync_copy(rbuf, out_hbm.at[pl.ds(rb, R), 0, pl.ds(c * DH, DH)])
            return 0
        lax.fori_loop(0, nrc, ego, 0)

        plsc.subcore_barrier()

        hop_srcs = [agg0_hbm, scr1_hbm, scr2_hbm]
        hop_raws = [scr1_hbm, scr2_hbm, None]

        for h in range(1, N_HOPS + 1):
            src = hop_srcs[h - 1]
            raw = hop_raws[h - 1]

            def edge_chunk(i, _, src=src):
                eb = ebase + i * E
                pltpu.sync_copy(erows_hbm.at[pl.ds(eb, E)], rows_v)
                pltpu.sync_copy(ecols_hbm.at[pl.ds(eb, E)], cols_v)
                pltpu.sync_copy(evals_hbm.at[pl.ds(eb, E)], vals_v)
                pltpu.async_copy(src.at[c].at[cols_v], gbuf, sem).wait()

                _scale_rows(gbuf, E, vals_v, 0)

                pltpu.sync_copy(gbuf, acc.at[rows_v], add=True)
                return 0
            lax.fori_loop(0, NCHUNK, edge_chunk, 0)

            plsc.subcore_barrier()

            pltpu.sync_copy(coef_hbm.at[h, pl.ds(row_base, ROWS_BIG)], coef_v)

            def drain(j, _, raw=raw, h=h):
                rb = row_base + j * R
                pltpu.sync_copy(acc.at[pl.ds(rb, R)], rbuf)
                if raw is not None:
                    pltpu.sync_copy(rbuf, raw.at[c, pl.ds(rb, R)])
                    pltpu.sync_copy(zbuf, acc.at[pl.ds(rb, R)])
                _scale_rows(rbuf, R, coef_v, j * R)
                pltpu.sync_copy(rbuf, out_hbm.at[pl.ds(rb, R), h, pl.ds(c * DH, DH)])
                return 0
            lax.fori_loop(0, nrc, drain, 0)

            plsc.subcore_barrier()

    f = pl.kernel(
        body,
        out_type=(
            jax.ShapeDtypeStruct((N, N_HOPS + 1, D), jnp.float32),
            jax.ShapeDtypeStruct((NC, N, DH), jnp.float32),
            jax.ShapeDtypeStruct((NC, N, DH), jnp.float32),
        ),
        mesh=mesh,
        scratch_types=[
            pltpu.VMEM_SHARED((N, DH), jnp.float32),
            pltpu.VMEM((E,), jnp.int32),
            pltpu.VMEM((E,), jnp.float32),
            pltpu.VMEM((E,), jnp.int32),
            pltpu.VMEM((E, DH), jnp.float32),
            pltpu.VMEM((R, DH), jnp.float32),
            pltpu.VMEM((R, DH), jnp.float32),
            pltpu.VMEM((ROWS_BIG,), jnp.float32),
            pltpu.SemaphoreType.DMA,
        ],
    )
    out, _, _ = f(agg0, coefs, erows, ecols, evals)
    return out


def kernel(user_embed, item_embed, user_t, item_t, edge_rows, edge_cols, edge_vals):
    all_embed = jnp.concatenate([user_embed, item_embed], axis=0)
    agg0 = all_embed.reshape(N, NC, DH).transpose(1, 0, 2)

    t = jnp.concatenate([user_t[:, 0], item_t[:, 0]])
    decay = 1.0 - t
    coefs = jnp.stack([t, t * decay, t * decay**2, t * decay**3])
    coefs = jnp.pad(coefs, ((0, 0), (0, COEF_PAD - N)))

    out = _graph_conv(agg0, coefs, edge_rows, edge_cols, edge_vals)
    return out[:N_USERS], out[N_USERS:]
